# baseline (speedup 1.0000x reference)
import jax
import jax.numpy as jnp
from jax.experimental import pallas as pl
from jax.experimental.pallas import tpu as pltpu
from jax.experimental.pallas import tpu_sc as plsc

RATIO = 0.25
R = 0.3
K = 32

_N = 16384
_M = 4096
_GR = 128
_QR = _M // 128
_QB = 256
_PB = 8192
_NPAIR = _M * K


def _fps_kernel(x_ref, y_ref, z_ref, qx_ref, qy_ref, qz_ref):
    xv = x_ref[...]
    yv = y_ref[...]
    zv = z_ref[...]
    row = jax.lax.broadcasted_iota(jnp.int32, (_GR, 128), 0)
    col = jax.lax.broadcasted_iota(jnp.int32, (_GR, 128), 1)
    idx2d = row * 128 + col
    qrow = jax.lax.broadcasted_iota(jnp.int32, (_QR, 128), 0)
    qcol = jax.lax.broadcasted_iota(jnp.int32, (_QR, 128), 1)
    qidx2d = qrow * 128 + qcol

    lcol = jax.lax.broadcasted_iota(jnp.int32, (1, 128), 1)

    def body(i, state):
        dmin, cur = state
        r = jax.lax.shift_right_logical(cur, 7)
        lane = jax.lax.bitwise_and(cur, 127)
        cmask = lcol == lane
        cx = jnp.sum(jnp.where(cmask, x_ref[pl.ds(r, 1), :], 0.0))
        cy = jnp.sum(jnp.where(cmask, y_ref[pl.ds(r, 1), :], 0.0))
        cz = jnp.sum(jnp.where(cmask, z_ref[pl.ds(r, 1), :], 0.0))
        dx = xv - cx
        dy = yv - cy
        dz = zv - cz
        d = (dx * dx + dy * dy) + dz * dz
        dmin = jnp.minimum(dmin, d)
        mx = jnp.max(dmin)
        nxt = jnp.min(jnp.where(dmin == mx, idx2d, _N))
        qmask = qidx2d == i
        qx_ref[...] = jnp.where(qmask, cx, qx_ref[...])
        qy_ref[...] = jnp.where(qmask, cy, qy_ref[...])
        qz_ref[...] = jnp.where(qmask, cz, qz_ref[...])
        return dmin, nxt

    dmin0 = jnp.full((_GR, 128), jnp.inf, dtype=jnp.float32)
    jax.lax.fori_loop(0, _M, body, (dmin0, jnp.int32(0)))


def _fps_q(pos):
    x = pos[:, 0].reshape(_GR, 128)
    y = pos[:, 1].reshape(_GR, 128)
    z = pos[:, 2].reshape(_GR, 128)
    qx, qy, qz = pl.pallas_call(
        _fps_kernel,
        out_shape=[jax.ShapeDtypeStruct((_QR, 128), jnp.float32)] * 3,
    )(x, y, z)
    return jnp.stack(
        [qx.reshape(_M), qy.reshape(_M), qz.reshape(_M)], axis=-1
    )


def _topk_kernel(q_ref, pT_ref, idx_ref, d2v_ref, d2_scr):
    qv = q_ref[...]
    pT = pT_ref[...]
    qn = jnp.sum(qv * qv, axis=1, keepdims=True)
    pn = jnp.sum(pT * pT, axis=0, keepdims=True)
    dot = jnp.dot(qv, pT, preferred_element_type=jnp.float32)
    d2_scr[...] = (qn + pn) - 2.0 * dot
    j2d = jax.lax.broadcasted_iota(jnp.int32, (_QB, _N), 1)
    inf = jnp.inf
    i1p = jnp.full((_QB, 1), -1, jnp.int32)
    i2p = i1p
    for s in range(K // 2):
        v = d2_scr[...]
        v = jnp.where((j2d == i1p) | (j2d == i2p), inf, v)
        d2_scr[...] = v
        m1 = jnp.min(v, axis=1, keepdims=True)
        v = d2_scr[...]
        eq = v == m1
        i1 = jnp.min(jnp.where(eq, j2d, _N), axis=1, keepdims=True)
        c1 = jnp.sum(jnp.where(eq, 1.0, 0.0), axis=1, keepdims=True)
        s2 = jnp.min(jnp.where(eq, inf, v), axis=1, keepdims=True)
        m2 = jnp.where(c1 > 1.0, m1, s2)
        v = d2_scr[...]
        cand2 = jnp.where((v == m2) & (j2d != i1), j2d, _N)
        i2 = jnp.min(cand2, axis=1, keepdims=True)
        idx_ref[:, 2 * s : 2 * s + 1] = i1
        idx_ref[:, 2 * s + 1 : 2 * s + 2] = i2
        d2v_ref[:, 2 * s : 2 * s + 1] = m1
        d2v_ref[:, 2 * s + 1 : 2 * s + 2] = m2
        i1p, i2p = i1, i2


def _topk(q, posc):
    pT = posc.T
    nblk = _M // _QB
    idx, d2v = pl.pallas_call(
        _topk_kernel,
        grid=(nblk,),
        in_specs=[
            pl.BlockSpec((_QB, 3), lambda i: (i, 0)),
            pl.BlockSpec((3, _N), lambda i: (0, 0)),
        ],
        out_specs=[
            pl.BlockSpec((_QB, K), lambda i: (i, 0)),
            pl.BlockSpec((_QB, K), lambda i: (i, 0)),
        ],
        out_shape=[
            jax.ShapeDtypeStruct((_M, K), jnp.int32),
            jax.ShapeDtypeStruct((_M, K), jnp.float32),
        ],
        scratch_shapes=[pltpu.VMEM((_QB, _N), jnp.float32)],
    )(q, pT)
    return idx, d2v


_GD = 128
_NW = 32
_CHUNK = 512


def _sc_gather(posP, idxf):
    b_per_w = _NPAIR // _NW
    mesh = plsc.VectorSubcoreMesh(
        core_axis_name="c", subcore_axis_name="s"
    )

    @pl.kernel(
        out_type=jax.ShapeDtypeStruct((_NPAIR, _GD), jnp.float32),
        mesh=mesh,
        scratch_types=[
            pltpu.VMEM((_CHUNK,), jnp.int32),
            pltpu.VMEM((_CHUNK, _GD), jnp.float32),
            pltpu.SemaphoreType.DMA,
        ],
    )
    def gk(x_hbm, i_hbm, o_hbm, idx_v, rows_v, sem):
        wid = jax.lax.axis_index("s") * 2 + jax.lax.axis_index("c")
        base = wid * b_per_w

        @pl.loop(0, b_per_w, step=_CHUNK)
        def _(off):
            pltpu.sync_copy(i_hbm.at[pl.ds(base + off, _CHUNK)], idx_v)
            pltpu.async_copy(x_hbm.at[idx_v], rows_v, sem).wait()
            pltpu.sync_copy(rows_v, o_hbm.at[pl.ds(base + off, _CHUNK)])

    return gk(posP, idxf)


def _mlp_kernel(
    pjT_ref, qrT_ref, d2v_ref, w1_ref, b1_ref, w2_ref, b2_ref,
    wc_ref, bc_ref, out_ref, acc_scr
):
    i = pl.program_id(0)
    rel = pjT_ref[0:3, :] - qrT_ref[0:3, :]
    h1 = jnp.dot(w1_ref[...], rel, preferred_element_type=jnp.float32)
    h1 = jnp.maximum(h1 + b1_ref[...], 0.0)
    h = jnp.dot(w2_ref[...], h1, preferred_element_type=jnp.float32)
    h = h + b2_ref[...]
    vrow = d2v_ref[0] <= (R * R)
    hm = jnp.where(vrow, h, -jnp.inf)
    bmax = jnp.max(hm, axis=1, keepdims=True)

    @pl.when(i == 0)
    def _():
        acc_scr[:, 0:1] = bmax

    @pl.when(i > 0)
    def _():
        acc_scr[:, 0:1] = jnp.maximum(acc_scr[:, 0:1], bmax)

    @pl.when(i == (_NPAIR // _PB) - 1)
    def _():
        pooled = acc_scr[:, 0:1]
        pooled = jnp.where(jnp.isfinite(pooled), pooled, 0.0)
        out = jnp.dot(
            wc_ref[...], pooled, preferred_element_type=jnp.float32
        )
        out_ref[...] = out + bc_ref[...]


def _mlp_max(pjT, qrT, d2v3, W1, b1, W2, b2, Wc, bc):
    nblk = _NPAIR // _PB
    full = lambda shape: pl.BlockSpec(shape, lambda i: tuple(0 for _ in shape))
    out = pl.pallas_call(
        _mlp_kernel,
        grid=(nblk,),
        in_specs=[
            pl.BlockSpec((8, _PB), lambda i: (0, i)),
            pl.BlockSpec((8, _PB), lambda i: (0, i)),
            pl.BlockSpec((1, 1, _PB), lambda i: (i, 0, 0)),
            full((32, 3)),
            full((32, 1)),
            full((32, 32)),
            full((32, 1)),
            full((10, 32)),
            full((10, 1)),
        ],
        out_specs=pl.BlockSpec((10, 1), lambda i: (0, 0)),
        out_shape=jax.ShapeDtypeStruct((10, 1), jnp.float32),
        scratch_shapes=[pltpu.VMEM((32, 128), jnp.float32)],
    )(
        pjT, qrT, d2v3,
        W1.T, b1.reshape(32, 1), W2.T, b2.reshape(32, 1),
        Wc.T, bc.reshape(10, 1),
    )
    return out


def kernel(pos, batch, W1, b1, W2, b2, Wc, bc):
    posc = jax.lax.stop_gradient(pos)
    q = _fps_q(posc)
    idx, d2v = _topk(q, posc)
    posP = jnp.pad(posc, ((0, 0), (0, _GD - 3)))
    pj = _sc_gather(posP, idx.reshape(_NPAIR))
    pjT = pj[:, :8].T
    qrT = jnp.repeat(
        jnp.pad(q, ((0, 0), (0, 5))).T, K, axis=1
    )
    d2v3 = d2v.reshape(_NPAIR // _PB, 1, _PB)
    out = _mlp_max(pjT, qrT, d2v3, W1, b1, W2, b2, Wc, bc)
    return out.T
